# Initial kernel scaffold; baseline (speedup 1.0000x reference)
import jax
import jax.numpy as jnp
from jax.experimental import pallas as pl


def kernel(x, w, w_sum, w_prod):
    raise NotImplementedError("write your pallas kernel here")


# speedup vs baseline: 64.9359x; 64.9359x over previous
import functools

import jax
import jax.numpy as jnp
from jax.experimental import pallas as pl
from jax.experimental.pallas import tpu as pltpu

N_POLY = 3
SEGMENTS = 64
IN_FEATURES = 64
OUT_FEATURES = 64
N_WEIGHTS = (N_POLY - 1) * SEGMENTS + 1
LENGTH = 2.0
HALF = 0.5 * LENGTH


def _body(x_ref, w_ref, ws_ref, wp_ref, o_ref, sum_ref, prod_ref):
    i = pl.program_id(0)
    b = x_ref.shape[2]
    xv = x_ref[0]
    idf = (xv + HALF) / LENGTH * SEGMENTS
    idi = jnp.clip(idf.astype(jnp.int32), 0, SEGMENTS - 1)
    idff = idi.astype(jnp.float32)
    x_min = idff / SEGMENTS * 2.0 - 1.0
    x_max = (idff + 1.0) / SEGMENTS * 2.0 - 1.0
    x_in = LENGTH * ((xv - x_min) / (x_max - x_min)) - HALF
    c0 = 0.5 * x_in * (x_in - 1.0)
    c1 = 1.0 - x_in * x_in
    c2 = 0.5 * x_in * (x_in + 1.0)

    k = jax.lax.broadcasted_iota(jnp.int32, (N_WEIGHTS, b), 0)
    d = k - 2 * idi
    zero = jnp.zeros_like(d, dtype=jnp.float32)
    c_mat = (
        jnp.where(d == 0, c0, zero)
        + jnp.where(d == 1, c1, zero)
        + jnp.where(d == 2, c2, zero)
    )
    a = jax.lax.dot_general(
        c_mat,
        w_ref[0],
        dimension_numbers=(((0,), (0,)), ((), ())),
        preferred_element_type=jnp.float32,
    )

    @pl.when(i == 0)
    def _():
        sum_ref[...] = a
        prod_ref[...] = a

    @pl.when(i > 0)
    def _():
        sum_ref[...] += a
        prod_ref[...] *= a

    @pl.when(i == IN_FEATURES - 1)
    def _():
        o_ref[...] = sum_ref[...] * ws_ref[...] + prod_ref[...] * wp_ref[...]


@jax.jit
def kernel(x, w, w_sum, w_prod):
    b = x.shape[0]
    x_t = jnp.transpose(x).reshape(IN_FEATURES, 1, b)
    w_t = jnp.transpose(w, (1, 2, 0))
    ws = w_sum.reshape(1, OUT_FEATURES)
    wp = w_prod.reshape(1, OUT_FEATURES)
    out = pl.pallas_call(
        _body,
        grid=(IN_FEATURES,),
        in_specs=[
            pl.BlockSpec((1, 1, b), lambda i: (i, 0, 0)),
            pl.BlockSpec((1, N_WEIGHTS, OUT_FEATURES), lambda i: (i, 0, 0)),
            pl.BlockSpec((1, OUT_FEATURES), lambda i: (0, 0)),
            pl.BlockSpec((1, OUT_FEATURES), lambda i: (0, 0)),
        ],
        out_specs=pl.BlockSpec((b, OUT_FEATURES), lambda i: (0, 0)),
        out_shape=jax.ShapeDtypeStruct((b, OUT_FEATURES), jnp.float32),
        scratch_shapes=[
            pltpu.VMEM((b, OUT_FEATURES), jnp.float32),
            pltpu.VMEM((b, OUT_FEATURES), jnp.float32),
        ],
    )(x_t, w_t, ws, wp)
    return out
